# initial kernel scaffold (unmeasured)
import jax
import jax.numpy as jnp
from jax import lax
from jax.experimental import pallas as pl
from jax.experimental.pallas import tpu as pltpu

N_DEV = 4
B, SQ, SKV, DM = 2, 256, 256, 512
H_LOC, DH = 4, 64
BF = jnp.bfloat16

_sem_signal = getattr(pl, "semaphore_signal", None) or pltpu.semaphore_signal
_sem_wait = getattr(pl, "semaphore_wait", None) or pltpu.semaphore_wait
_DevId = getattr(pl, "DeviceIdType", None) or pltpu.DeviceIdType


def kernel(x, Wq, K_ext, V_ext, Wo):
    def body(x_ref, wq_ref, k_ref, v_ref, wo_ref, out_ref,
             sbuf, rbuf, cbuf,
             scat_send_sems, scat_recv_sem, ring_send_sems, ring_recv_sems):
        my = lax.axis_index("i")
        right = lax.rem(my + 1, N_DEV)

        barrier = pltpu.get_barrier_semaphore()
        for d in range(N_DEV):
            @pl.when(my != d)
            def _(d=d):
                _sem_signal(barrier, inc=1, device_id=(d,),
                            device_id_type=_DevId.MESH)
        _sem_wait(barrier, N_DEV - 1)

        @pl.when(my == 0)
        def _():
            for j in range(1, N_DEV):
                hs = j * H_LOC
                for b in range(B):
                    sbuf[j - 1, 0, b] = k_ref[b, :, hs:hs + H_LOC, :].astype(BF)
                    sbuf[j - 1, 1, b] = v_ref[b, :, hs:hs + H_LOC, :].astype(BF)
            for j in range(1, N_DEV):
                rdma = pltpu.make_async_remote_copy(
                    src_ref=sbuf.at[j - 1], dst_ref=rbuf,
                    send_sem=scat_send_sems.at[j - 1], recv_sem=scat_recv_sem,
                    device_id=(j,), device_id_type=_DevId.MESH,
                )
                rdma.start()
            for b in range(B):
                rbuf[0, b] = k_ref[b, :, 0:H_LOC, :].astype(BF)
                rbuf[1, b] = v_ref[b, :, 0:H_LOC, :].astype(BF)

        wq16 = wq_ref[:].astype(BF)
        wo16 = wo_ref[:].astype(BF)
        q = [jnp.dot(x_ref[b].astype(BF), wq16,
                     preferred_element_type=jnp.float32) for b in range(B)]

        rowb = lax.broadcasted_iota(jnp.int32, (SQ, SKV), 0) // 64
        colb = lax.broadcasted_iota(jnp.int32, (SQ, SKV), 1) // 64
        mask = colb <= rowb

        @pl.when(my != 0)
        def _():
            rdma = pltpu.make_async_remote_copy(
                src_ref=sbuf.at[0], dst_ref=rbuf,
                send_sem=scat_send_sems.at[0], recv_sem=scat_recv_sem,
                device_id=(0,), device_id_type=_DevId.MESH,
            )
            rdma.wait_recv()

        for b in range(B):
            kb = rbuf[0, b]
            vb = rbuf[1, b]
            acc = jnp.zeros((SQ, DM), jnp.float32)
            for hh in range(H_LOC):
                qh = q[b][:, hh * DH:(hh + 1) * DH].astype(BF)
                kh = kb[:, hh, :]
                vh = vb[:, hh, :]
                s = lax.dot_general(qh, kh, (((1,), (1,)), ((), ())),
                                    preferred_element_type=jnp.float32) * 0.125
                s = jnp.where(mask, s, -1e9)
                m = jnp.max(s, axis=1, keepdims=True)
                w = jnp.exp(s - m)
                w = w / jnp.sum(w, axis=1, keepdims=True)
                ctx = jnp.dot(w.astype(BF), vh,
                              preferred_element_type=jnp.float32)
                acc = acc + jnp.dot(ctx.astype(BF),
                                    wo16[hh * DH:(hh + 1) * DH, :],
                                    preferred_element_type=jnp.float32)
            out_ref[b] = acc

        @pl.when(my == 0)
        def _():
            for j in range(1, N_DEV):
                rdma = pltpu.make_async_remote_copy(
                    src_ref=sbuf.at[j - 1], dst_ref=rbuf,
                    send_sem=scat_send_sems.at[j - 1], recv_sem=scat_recv_sem,
                    device_id=(j,), device_id_type=_DevId.MESH,
                )
                rdma.wait_send()

        for b in range(B):
            cbuf[0, b] = out_ref[b].astype(BF)
        for h in range(N_DEV - 1):
            ss, rs = h % 2, (h + 1) % 2
            rdma = pltpu.make_async_remote_copy(
                src_ref=cbuf.at[ss], dst_ref=cbuf.at[rs],
                send_sem=ring_send_sems.at[ss], recv_sem=ring_recv_sems.at[rs],
                device_id=(right,), device_id_type=_DevId.MESH,
            )
            rdma.start()
            rdma.wait()
            for b in range(B):
                out_ref[b] = out_ref[b] + cbuf[rs, b].astype(jnp.float32)

    return pl.pallas_call(
        body,
        out_shape=jax.ShapeDtypeStruct((B, SQ, DM), jnp.float32),
        in_specs=[pl.BlockSpec(memory_space=pltpu.VMEM)] * 5,
        out_specs=pl.BlockSpec(memory_space=pltpu.VMEM),
        scratch_shapes=[
            pltpu.VMEM((N_DEV - 1, 2, B, SKV, H_LOC, DH), BF),
            pltpu.VMEM((2, B, SKV, H_LOC, DH), BF),
            pltpu.VMEM((2, B, SQ, DM), BF),
            pltpu.SemaphoreType.DMA((N_DEV - 1,)),
            pltpu.SemaphoreType.DMA,
            pltpu.SemaphoreType.DMA((2,)),
            pltpu.SemaphoreType.DMA((2,)),
        ],
        compiler_params=pltpu.CompilerParams(collective_id=0),
    )(x, Wq, K_ext, V_ext, Wo)


# baseline (device time: 60977 ns/iter reference)
import jax
import jax.numpy as jnp
from jax import lax
from jax.experimental import pallas as pl
from jax.experimental.pallas import tpu as pltpu

N_DEV = 4
B, SQ, SKV, DM = 2, 256, 256, 512
H_LOC, DH = 4, 64
BF = jnp.bfloat16

_sem_signal = getattr(pl, "semaphore_signal", None) or pltpu.semaphore_signal
_sem_wait = getattr(pl, "semaphore_wait", None) or pltpu.semaphore_wait
_DevId = getattr(pl, "DeviceIdType", None) or pltpu.DeviceIdType


def kernel(x, Wq, K_ext, V_ext, Wo):
    def body(x_ref, wq_ref, k_ref, v_ref, wo_ref, out_ref,
             sbuf, rbuf, cbuf,
             scat_send_sems, scat_recv_sem, ring_send_sems, ring_recv_sems):
        my = lax.axis_index("i")
        right = lax.rem(my + 1, N_DEV)

        barrier = pltpu.get_barrier_semaphore()
        for d in range(N_DEV):
            @pl.when(my != d)
            def _(d=d):
                _sem_signal(barrier, inc=1, device_id=(d,),
                            device_id_type=_DevId.MESH)
        _sem_wait(barrier, N_DEV - 1)

        @pl.when(my == 0)
        def _():
            for j in range(1, N_DEV):
                hs = j * H_LOC
                for b in range(B):
                    sbuf[j - 1, 0, b] = k_ref[b, :, hs:hs + H_LOC, :].astype(BF)
                    sbuf[j - 1, 1, b] = v_ref[b, :, hs:hs + H_LOC, :].astype(BF)
            for j in range(1, N_DEV):
                rdma = pltpu.make_async_remote_copy(
                    src_ref=sbuf.at[j - 1], dst_ref=rbuf,
                    send_sem=scat_send_sems.at[j - 1], recv_sem=scat_recv_sem,
                    device_id=(j,), device_id_type=_DevId.MESH,
                )
                rdma.start()
            for b in range(B):
                rbuf[0, b] = k_ref[b, :, 0:H_LOC, :].astype(BF)
                rbuf[1, b] = v_ref[b, :, 0:H_LOC, :].astype(BF)

        wq16 = wq_ref[:].astype(BF)
        wo16 = wo_ref[:].astype(BF)
        q = [jnp.dot(x_ref[b].astype(BF), wq16,
                     preferred_element_type=jnp.float32) for b in range(B)]

        rowb = lax.broadcasted_iota(jnp.int32, (SQ, SKV), 0) // 64
        colb = lax.broadcasted_iota(jnp.int32, (SQ, SKV), 1) // 64
        mask = colb <= rowb

        @pl.when(my != 0)
        def _():
            rdma = pltpu.make_async_remote_copy(
                src_ref=sbuf.at[0], dst_ref=rbuf,
                send_sem=scat_send_sems.at[0], recv_sem=scat_recv_sem,
                device_id=(0,), device_id_type=_DevId.MESH,
            )
            rdma.wait_recv()

        for b in range(B):
            kb = rbuf[0, b]
            vb = rbuf[1, b]
            acc = jnp.zeros((SQ, DM), jnp.float32)
            for hh in range(H_LOC):
                qh = q[b][:, hh * DH:(hh + 1) * DH].astype(BF)
                kh = kb[:, hh, :]
                vh = vb[:, hh, :]
                s = lax.dot_general(qh, kh, (((1,), (1,)), ((), ())),
                                    preferred_element_type=jnp.float32) * 0.125
                s = jnp.where(mask, s, -1e9)
                m = jnp.max(s, axis=1, keepdims=True)
                w = jnp.exp(s - m)
                w = w / jnp.sum(w, axis=1, keepdims=True)
                ctx = jnp.dot(w.astype(BF), vh,
                              preferred_element_type=jnp.float32)
                acc = acc + jnp.dot(ctx.astype(BF),
                                    wo16[hh * DH:(hh + 1) * DH, :],
                                    preferred_element_type=jnp.float32)
            out_ref[b] = acc

        @pl.when(my == 0)
        def _():
            for j in range(1, N_DEV):
                rdma = pltpu.make_async_remote_copy(
                    src_ref=sbuf.at[j - 1], dst_ref=rbuf,
                    send_sem=scat_send_sems.at[j - 1], recv_sem=scat_recv_sem,
                    device_id=(j,), device_id_type=_DevId.MESH,
                )
                rdma.wait_send()

        for b in range(B):
            cbuf[0, b] = out_ref[b].astype(BF)
        for h in range(N_DEV - 1):
            rdma = pltpu.make_async_remote_copy(
                src_ref=cbuf.at[h], dst_ref=cbuf.at[h + 1],
                send_sem=ring_send_sems.at[h], recv_sem=ring_recv_sems.at[h],
                device_id=(right,), device_id_type=_DevId.MESH,
            )
            rdma.start()
            rdma.wait()
            for b in range(B):
                out_ref[b] = out_ref[b] + cbuf[h + 1, b].astype(jnp.float32)

    return pl.pallas_call(
        body,
        out_shape=jax.ShapeDtypeStruct((B, SQ, DM), jnp.float32),
        in_specs=[pl.BlockSpec(memory_space=pltpu.VMEM)] * 5,
        out_specs=pl.BlockSpec(memory_space=pltpu.VMEM),
        scratch_shapes=[
            pltpu.VMEM((N_DEV - 1, 2, B, SKV, H_LOC, DH), BF),
            pltpu.VMEM((2, B, SKV, H_LOC, DH), BF),
            pltpu.VMEM((N_DEV, B, SQ, DM), BF),
            pltpu.SemaphoreType.DMA((N_DEV - 1,)),
            pltpu.SemaphoreType.DMA,
            pltpu.SemaphoreType.DMA((N_DEV - 1,)),
            pltpu.SemaphoreType.DMA((N_DEV - 1,)),
        ],
        compiler_params=pltpu.CompilerParams(collective_id=0),
    )(x, Wq, K_ext, V_ext, Wo)


# device time: 49980 ns/iter; 1.2200x vs baseline; 1.2200x over previous
import jax
import jax.numpy as jnp
from jax import lax
from jax.experimental import pallas as pl
from jax.experimental.pallas import tpu as pltpu

N_DEV = 4
B, SQ, SKV, DM = 2, 256, 256, 512
H_LOC, DH = 4, 64
BF = jnp.bfloat16

_sem_signal = getattr(pl, "semaphore_signal", None) or pltpu.semaphore_signal
_sem_wait = getattr(pl, "semaphore_wait", None) or pltpu.semaphore_wait
_DevId = getattr(pl, "DeviceIdType", None) or pltpu.DeviceIdType


def kernel(x, Wq, K_ext, V_ext, Wo):
    def body(x_ref, wq_ref, k_ref, v_ref, wo_ref, out_ref,
             sk, sv, rk, rv, cbuf,
             k_send_sems, v_send_sems, k_recv_sem, v_recv_sem,
             ar_send_sems, ar_recv_sems):
        my = lax.axis_index("i")

        barrier = pltpu.get_barrier_semaphore()
        for d in range(N_DEV):
            @pl.when(my != d)
            def _(d=d):
                _sem_signal(barrier, inc=1, device_id=(d,),
                            device_id_type=_DevId.MESH)
        _sem_wait(barrier, N_DEV - 1)

        def scat_rdma(src, dst, sems, rsem, j):
            return pltpu.make_async_remote_copy(
                src_ref=src.at[j - 1], dst_ref=dst,
                send_sem=sems.at[j - 1], recv_sem=rsem,
                device_id=(j,), device_id_type=_DevId.MESH,
            )

        @pl.when(my == 0)
        def _():
            for j in range(1, N_DEV):
                hs = j * H_LOC
                for b in range(B):
                    sk[j - 1, b] = k_ref[b, :, hs:hs + H_LOC, :].astype(BF)
            for j in range(1, N_DEV):
                scat_rdma(sk, rk, k_send_sems, k_recv_sem, j).start()
            for j in range(1, N_DEV):
                hs = j * H_LOC
                for b in range(B):
                    sv[j - 1, b] = v_ref[b, :, hs:hs + H_LOC, :].astype(BF)
            for j in range(1, N_DEV):
                scat_rdma(sv, rv, v_send_sems, v_recv_sem, j).start()
            for b in range(B):
                rk[b] = k_ref[b, :, 0:H_LOC, :].astype(BF)
                rv[b] = v_ref[b, :, 0:H_LOC, :].astype(BF)

        wq16 = wq_ref[:].astype(BF)
        wo16 = wo_ref[:].astype(BF)
        q = [jnp.dot(x_ref[b].astype(BF), wq16,
                     preferred_element_type=jnp.float32) for b in range(B)]

        rowb = lax.broadcasted_iota(jnp.int32, (SQ, SKV), 0) // 64
        colb = lax.broadcasted_iota(jnp.int32, (SQ, SKV), 1) // 64
        mask = colb <= rowb

        @pl.when(my != 0)
        def _():
            scat_rdma(sk, rk, k_send_sems, k_recv_sem, 1).wait_recv()

        ws = []
        for b in range(B):
            kb = rk[b]
            for hh in range(H_LOC):
                qh = q[b][:, hh * DH:(hh + 1) * DH].astype(BF)
                kh = kb[:, hh, :]
                s = lax.dot_general(qh, kh, (((1,), (1,)), ((), ())),
                                    preferred_element_type=jnp.float32) * 0.125
                s = jnp.where(mask, s, -1e9)
                m = jnp.max(s, axis=1, keepdims=True)
                w = jnp.exp(s - m)
                ws.append((w / jnp.sum(w, axis=1, keepdims=True)).astype(BF))

        @pl.when(my != 0)
        def _():
            scat_rdma(sv, rv, v_send_sems, v_recv_sem, 1).wait_recv()

        for b in range(B):
            vb = rv[b]
            acc = jnp.zeros((SQ, DM), jnp.float32)
            for hh in range(H_LOC):
                ctx = jnp.dot(ws[b * H_LOC + hh], vb[:, hh, :],
                              preferred_element_type=jnp.float32)
                acc = acc + jnp.dot(ctx.astype(BF),
                                    wo16[hh * DH:(hh + 1) * DH, :],
                                    preferred_element_type=jnp.float32)
            cbuf[N_DEV - 1, b] = acc.astype(BF)

        for o in range(1, N_DEV):
            tgt = lax.rem(my + o, N_DEV)
            pltpu.make_async_remote_copy(
                src_ref=cbuf.at[N_DEV - 1], dst_ref=cbuf.at[o - 1],
                send_sem=ar_send_sems.at[o - 1], recv_sem=ar_recv_sems.at[o - 1],
                device_id=(tgt,), device_id_type=_DevId.MESH,
            ).start()
        for o in range(1, N_DEV):
            tgt = lax.rem(my + o, N_DEV)
            pltpu.make_async_remote_copy(
                src_ref=cbuf.at[N_DEV - 1], dst_ref=cbuf.at[o - 1],
                send_sem=ar_send_sems.at[o - 1], recv_sem=ar_recv_sems.at[o - 1],
                device_id=(tgt,), device_id_type=_DevId.MESH,
            ).wait_recv()
        for b in range(B):
            out_ref[b] = (
                (cbuf[0, b].astype(jnp.float32) + cbuf[1, b].astype(jnp.float32))
                + (cbuf[2, b].astype(jnp.float32) + cbuf[3, b].astype(jnp.float32))
            )

        for o in range(1, N_DEV):
            tgt = lax.rem(my + o, N_DEV)
            pltpu.make_async_remote_copy(
                src_ref=cbuf.at[N_DEV - 1], dst_ref=cbuf.at[o - 1],
                send_sem=ar_send_sems.at[o - 1], recv_sem=ar_recv_sems.at[o - 1],
                device_id=(tgt,), device_id_type=_DevId.MESH,
            ).wait_send()
        @pl.when(my == 0)
        def _():
            for j in range(1, N_DEV):
                scat_rdma(sk, rk, k_send_sems, k_recv_sem, j).wait_send()
                scat_rdma(sv, rv, v_send_sems, v_recv_sem, j).wait_send()

    return pl.pallas_call(
        body,
        out_shape=jax.ShapeDtypeStruct((B, SQ, DM), jnp.float32),
        in_specs=[pl.BlockSpec(memory_space=pltpu.VMEM)] * 5,
        out_specs=pl.BlockSpec(memory_space=pltpu.VMEM),
        scratch_shapes=[
            pltpu.VMEM((N_DEV - 1, B, SKV, H_LOC, DH), BF),
            pltpu.VMEM((N_DEV - 1, B, SKV, H_LOC, DH), BF),
            pltpu.VMEM((B, SKV, H_LOC, DH), BF),
            pltpu.VMEM((B, SKV, H_LOC, DH), BF),
            pltpu.VMEM((N_DEV, B, SQ, DM), BF),
            pltpu.SemaphoreType.DMA((N_DEV - 1,)),
            pltpu.SemaphoreType.DMA((N_DEV - 1,)),
            pltpu.SemaphoreType.DMA,
            pltpu.SemaphoreType.DMA,
            pltpu.SemaphoreType.DMA((N_DEV - 1,)),
            pltpu.SemaphoreType.DMA((N_DEV - 1,)),
        ],
        compiler_params=pltpu.CompilerParams(collective_id=0),
    )(x, Wq, K_ext, V_ext, Wo)
